# baseline (speedup 1.0000x reference)
import functools

import jax
import jax.numpy as jnp
from jax import lax
from jax.experimental import pallas as pl
from jax.experimental.pallas import tpu as pltpu
from jax.experimental.pallas import tpu_sc as plsc

_NC = 2
_NS = 16
_NW = _NC * _NS


def _build(B0, B1, V, D):
    bw = B0 // _NW
    mesh = plsc.VectorSubcoreMesh(core_axis_name="c", subcore_axis_name="s")

    @functools.partial(
        pl.kernel,
        mesh=mesh,
        out_type=jax.ShapeDtypeStruct((B1, B0, D), jnp.float32),
        compiler_params=pltpu.CompilerParams(use_tc_tiling_on_sc=False),
        scratch_types=[
            pltpu.VMEM((bw,), jnp.int32),
            pltpu.VMEM((bw,), jnp.int32),
            pltpu.VMEM((bw, D), jnp.float32),
            pltpu.VMEM((bw, D), jnp.float32),
            pltpu.SemaphoreType.DMA,
            pltpu.SemaphoreType.DMA,
            pltpu.SemaphoreType.DMA,
            pltpu.SemaphoreType.DMA,
        ],
    )
    def k(idxT_hbm, table_hbm, outT_hbm, ia, ib, ba, bb, gsa, gsb, wsa, wsb):
        wid = lax.axis_index("s") * _NC + lax.axis_index("c")
        b0 = wid * bw

        def stage(s, idx_v, buf, gsem, wsem):
            pltpu.sync_copy(idxT_hbm.at[s, pl.ds(b0, bw)], idx_v)
            pltpu.async_copy(table_hbm.at[idx_v], buf, gsem).wait()
            return pltpu.async_copy(buf, outT_hbm.at[s, pl.ds(b0, bw)], wsem)

        wa = stage(0, ia, ba, gsa, wsa)
        wb = stage(1, ib, bb, gsb, wsb)

        def body(i, carry):
            s = 2 * i
            wa.wait()
            stage(s, ia, ba, gsa, wsa)
            wb.wait()
            stage(s + 1, ib, bb, gsb, wsb)
            return carry

        lax.fori_loop(1, B1 // 2, body, 0)
        wa.wait()
        wb.wait()

    return k


def _build_tr(B0, B1, D):
    bw = B0 // _NW
    rw = bw * D // 128
    mesh = plsc.VectorSubcoreMesh(core_axis_name="c", subcore_axis_name="s")

    @functools.partial(
        pl.kernel,
        mesh=mesh,
        out_type=jax.ShapeDtypeStruct((B1, D, B0), jnp.float32),
        compiler_params=pltpu.CompilerParams(needs_layout_passes=False),
        scratch_types=[
            pltpu.VMEM((rw, 128), jnp.float32),
            pltpu.VMEM((rw, 128), jnp.float32),
            pltpu.VMEM((D, bw), jnp.float32),
            pltpu.VMEM((D, bw), jnp.float32),
            pltpu.SemaphoreType.DMA,
            pltpu.SemaphoreType.DMA,
        ],
    )
    def k2(x_hbm, y_hbm, xa, xb, oa, ob, wsa, wsb):
        wid = lax.axis_index("s") * _NC + lax.axis_index("c")
        b0 = wid * bw
        r0 = wid * rw
        lanes = lax.iota(jnp.int32, 16)
        epr = 128 // D
        rv = [j * 16 // epr + lanes // epr for j in range(bw // 16)]
        cv_base = (lanes % epr) * D

        def stage(s, xblk, obuf, wsem):
            pltpu.sync_copy(x_hbm.at[s, pl.ds(r0, rw)], xblk)
            def erow(e, c):
                cve = cv_base + e
                for g in range(0, bw // 16, 8):
                    vs = [
                        plsc.load_gather(xblk, [rv[g + j], cve])
                        for j in range(8)
                    ]
                    for j in range(8):
                        obuf[e, pl.ds((g + j) * 16, 16)] = vs[j]
                return c

            lax.fori_loop(0, D, erow, 0)
            return pltpu.async_copy(
                obuf, y_hbm.at[s, pl.ds(0, D), pl.ds(b0, bw)], wsem
            )

        wa = stage(0, xa, oa, wsa)
        wb = stage(1, xb, ob, wsb)

        def body(i, carry):
            s = 2 * i
            wa.wait()
            stage(s, xa, oa, wsa)
            wb.wait()
            stage(s + 1, xb, ob, wsb)
            return carry

        lax.fori_loop(1, B1 // 2, body, 0)
        wa.wait()
        wb.wait()

    return k2


def kernel(input, table):
    B0, B1 = input.shape
    V, D = table.shape
    outT = _build(B0, B1, V, D)(input.T.astype(jnp.int32), table)
    x3 = outT.reshape(B1, B0 * D // 128, 128)
    y = _build_tr(B0, B1, D)(x3)
    return y.transpose(2, 0, 1)


# speedup vs baseline: 1.9102x; 1.9102x over previous
import functools

import jax
import jax.numpy as jnp
from jax import lax
from jax.experimental import pallas as pl
from jax.experimental.pallas import tpu as pltpu
from jax.experimental.pallas import tpu_sc as plsc

_NC = 2
_NS = 16
_NW = _NC * _NS


def _build(B0, B1, V, D):
    bw = B0 // _NW
    mesh = plsc.VectorSubcoreMesh(core_axis_name="c", subcore_axis_name="s")

    @functools.partial(
        pl.kernel,
        mesh=mesh,
        out_type=jax.ShapeDtypeStruct((B1, B0, D), jnp.float32),
        compiler_params=pltpu.CompilerParams(use_tc_tiling_on_sc=False),
        scratch_types=[
            pltpu.VMEM((bw,), jnp.int32),
            pltpu.VMEM((bw,), jnp.int32),
            pltpu.VMEM((bw, D), jnp.float32),
            pltpu.VMEM((bw, D), jnp.float32),
            pltpu.SemaphoreType.DMA,
            pltpu.SemaphoreType.DMA,
            pltpu.SemaphoreType.DMA,
            pltpu.SemaphoreType.DMA,
        ],
    )
    def k(idxT_hbm, table_hbm, outT_hbm, ia, ib, ba, bb, gsa, gsb, wsa, wsb):
        wid = lax.axis_index("s") * _NC + lax.axis_index("c")
        b0 = wid * bw

        def stage(s, idx_v, buf, gsem, wsem):
            pltpu.sync_copy(idxT_hbm.at[s, pl.ds(b0, bw)], idx_v)
            pltpu.async_copy(table_hbm.at[idx_v], buf, gsem).wait()
            return pltpu.async_copy(buf, outT_hbm.at[s, pl.ds(b0, bw)], wsem)

        wa = stage(0, ia, ba, gsa, wsa)
        wb = stage(1, ib, bb, gsb, wsb)

        def body(i, carry):
            s = 2 * i
            wa.wait()
            stage(s, ia, ba, gsa, wsa)
            wb.wait()
            stage(s + 1, ib, bb, gsb, wsb)
            return carry

        lax.fori_loop(1, B1 // 2, body, 0)
        wa.wait()
        wb.wait()

    return k


def kernel(input, table):
    B0, B1 = input.shape
    V, D = table.shape
    outT = _build(B0, B1, V, D)(input.T.astype(jnp.int32), table)
    return outT.transpose(1, 0, 2)
